# baseline (speedup 1.0000x reference)
import functools

import jax
import jax.numpy as jnp
from jax import lax
from jax.experimental import pallas as pl
from jax.experimental.pallas import tpu as pltpu
from jax.experimental.pallas import tpu_sc as plsc

S, B, F = 40, 128, 4096
ST = 36
SSC = S - ST
T = ST * B
PE0 = (ST // 8) * 8

NC, NS = 2, 16
NW = NC * NS
SC_ROWS = SSC * B
RPW = SC_ROWS // NW
CH = 4
NCHUNK = RPW // CH
GROUPS = CH * F // 16
NBUF = 4
DEPTH = 2


def _sc_add(x2d, pe_table):
    mesh = plsc.VectorSubcoreMesh(core_axis_name="c", subcore_axis_name="s")

    @functools.partial(
        pl.kernel,
        out_type=jax.ShapeDtypeStruct((SC_ROWS, F), jnp.float32),
        mesh=mesh,
        scratch_types=[
            pltpu.VMEM((NBUF, CH, F), jnp.float32),
            pltpu.VMEM((S - PE0, F), jnp.float32),
            pltpu.SemaphoreType.DMA((NBUF,)),
            pltpu.SemaphoreType.DMA((NBUF,)),
            pltpu.SemaphoreType.DMA,
        ],
    )
    def k(x_hbm, pe_hbm, out_hbm, xbuf, pebuf, ldsem, stsem, pesem):
        wid = lax.axis_index("s") * NC + lax.axis_index("c")
        chunk0 = wid * NCHUNK
        lidx = (T + chunk0 * CH) // B - PE0

        pe_cp = pltpu.make_async_copy(pe_hbm.at[pl.ds(PE0, S - PE0)], pebuf, pesem)
        pe_cp.start()

        def load(i, slot):
            row0 = T + (chunk0 + i) * CH
            pltpu.make_async_copy(
                x_hbm.at[pl.ds(row0, CH)], xbuf.at[slot], ldsem.at[slot]
            ).start()

        def wait_load(slot):
            pltpu.make_async_copy(
                x_hbm.at[pl.ds(0, CH)], xbuf.at[slot], ldsem.at[slot]
            ).wait()

        def store(i, slot):
            row0 = (chunk0 + i) * CH
            pltpu.make_async_copy(
                xbuf.at[slot], out_hbm.at[pl.ds(row0, CH)], stsem.at[slot]
            ).start()

        def wait_store(slot):
            pltpu.make_async_copy(
                xbuf.at[slot], out_hbm.at[pl.ds(0, CH)], stsem.at[slot]
            ).wait()

        for i in range(min(DEPTH, NCHUNK)):
            load(i, i)
        pe_cp.wait()

        for i in range(NCHUNK):
            slot = i % NBUF
            wait_load(slot)
            nxt = i + DEPTH
            if nxt < NCHUNK:
                nslot = nxt % NBUF
                if nxt >= NBUF:
                    wait_store(nslot)
                load(nxt, nslot)
            @plsc.parallel_loop(0, GROUPS, 1, unroll=8)
            def body(g):
                r = g >> 8
                col = (g & 255) * 16
                xv = xbuf[slot, r, pl.ds(col, 16)]
                pv = pebuf[lidx, pl.ds(col, 16)]
                xbuf[slot, r, pl.ds(col, 16)] = xv + pv

            store(i, slot)

        for i in range(max(NCHUNK - NBUF, 0), NCHUNK):
            wait_store(i % NBUF)

    return k(x2d, pe_table)


BB = 32


def _tc_body(x_ref, pe_ref, o_ref):
    i = pl.program_id(0)
    o_ref[...] = x_ref[...] + pe_ref[pl.ds(i, 1), :]


def _tc_add(x, pe_table):
    return pl.pallas_call(
        _tc_body,
        grid=(ST, B // BB),
        in_specs=[
            pl.BlockSpec((1, BB, F), lambda i, j: (i, j, 0)),
            pl.BlockSpec((S, F), lambda i, j: (0, 0)),
        ],
        out_specs=pl.BlockSpec((1, BB, F), lambda i, j: (i, j, 0)),
        out_shape=jax.ShapeDtypeStruct((S, B, F), x.dtype),
    )(x, pe_table)


def kernel(x, pe_table):
    sc_part = _sc_add(x.reshape(S * B, F), pe_table)
    tc_full = _tc_add(x, pe_table)
    return lax.dynamic_update_slice(
        tc_full, sc_part.reshape(SSC, B, F), (ST, 0, 0)
    )


# speedup vs baseline: 1.7124x; 1.7124x over previous
import functools

import jax
import jax.numpy as jnp
from jax import lax
from jax.experimental import pallas as pl
from jax.experimental.pallas import tpu as pltpu
from jax.experimental.pallas import tpu_sc as plsc

S, B, F = 40, 128, 4096
ST = 36
SSC = S - ST
T = ST * B
PE0 = (ST // 8) * 8

NC, NS = 2, 16
NW = NC * NS
SC_ROWS = SSC * B
RPW = SC_ROWS // NW
CH = 4
NCHUNK = RPW // CH
GROUPS = CH * F // 16
NBUF = 4
DEPTH = 2


def _sc_add(x2d, pe_table):
    mesh = plsc.VectorSubcoreMesh(core_axis_name="c", subcore_axis_name="s")

    @functools.partial(
        pl.kernel,
        out_type=jax.ShapeDtypeStruct((SC_ROWS, F), jnp.float32),
        mesh=mesh,
        scratch_types=[
            pltpu.VMEM((NBUF, CH, F), jnp.float32),
            pltpu.VMEM((S - PE0, F), jnp.float32),
            pltpu.SemaphoreType.DMA((NBUF,)),
            pltpu.SemaphoreType.DMA((NBUF,)),
            pltpu.SemaphoreType.DMA,
        ],
    )
    def k(x_hbm, pe_hbm, out_hbm, xbuf, pebuf, ldsem, stsem, pesem):
        wid = lax.axis_index("s") * NC + lax.axis_index("c")
        chunk0 = wid * NCHUNK
        lidx = (T + chunk0 * CH) // B - PE0

        pe_cp = pltpu.make_async_copy(pe_hbm.at[pl.ds(PE0, S - PE0)], pebuf, pesem)
        pe_cp.start()

        def load(i, slot):
            row0 = T + (chunk0 + i) * CH
            pltpu.make_async_copy(
                x_hbm.at[pl.ds(row0, CH)], xbuf.at[slot], ldsem.at[slot]
            ).start()

        def wait_load(slot):
            pltpu.make_async_copy(
                x_hbm.at[pl.ds(0, CH)], xbuf.at[slot], ldsem.at[slot]
            ).wait()

        def store(i, slot):
            row0 = (chunk0 + i) * CH
            pltpu.make_async_copy(
                xbuf.at[slot], out_hbm.at[pl.ds(row0, CH)], stsem.at[slot]
            ).start()

        def wait_store(slot):
            pltpu.make_async_copy(
                xbuf.at[slot], out_hbm.at[pl.ds(0, CH)], stsem.at[slot]
            ).wait()

        for i in range(min(DEPTH, NCHUNK)):
            load(i, i)
        pe_cp.wait()

        for i in range(NCHUNK):
            slot = i % NBUF
            wait_load(slot)
            nxt = i + DEPTH
            if nxt < NCHUNK:
                nslot = nxt % NBUF
                if nxt >= NBUF:
                    wait_store(nslot)
                load(nxt, nslot)
            @plsc.parallel_loop(0, GROUPS, 1, unroll=8)
            def body(g):
                r = g >> 8
                col = (g & 255) * 16
                xv = xbuf[slot, r, pl.ds(col, 16)]
                pv = pebuf[lidx, pl.ds(col, 16)]
                xbuf[slot, r, pl.ds(col, 16)] = xv + pv

            store(i, slot)

        for i in range(max(NCHUNK - NBUF, 0), NCHUNK):
            wait_store(i % NBUF)

    return k(x2d, pe_table)


TCB = 6


def _tc_body(x_ref, pe_ref, o_ref):
    i = pl.program_id(0)
    for r in range(TCB):
        o_ref[r] = x_ref[r] + pe_ref[pl.ds(i * TCB + r, 1), :]


def _tc_add(x, pe_table):
    return pl.pallas_call(
        _tc_body,
        grid=(ST // TCB,),
        in_specs=[
            pl.BlockSpec((TCB, B, F), lambda i: (i, 0, 0)),
            pl.BlockSpec((S, F), lambda i: (0, 0)),
        ],
        out_specs=pl.BlockSpec((TCB, B, F), lambda i: (i, 0, 0)),
        out_shape=jax.ShapeDtypeStruct((S, B, F), x.dtype),
    )(x, pe_table)


def kernel(x, pe_table):
    sc_part = _sc_add(x.reshape(S * B, F), pe_table)
    tc_full = _tc_add(x, pe_table)
    return lax.dynamic_update_slice(
        tc_full, sc_part.reshape(SSC, B, F), (ST, 0, 0)
    )
